# baseline (speedup 1.0000x reference)
import functools

import jax
import jax.numpy as jnp
from jax import lax
from jax.experimental import pallas as pl
from jax.experimental.pallas import tpu as pltpu
from jax.experimental.pallas import tpu_sc as plsc

_EMBED = 128
_BLOCK = 2048
_BATCH = 4

_info = plsc.get_sparse_core_info()
_NC, _NS, _L = _info.num_cores, _info.num_subcores, _info.num_lanes
_NW = _NC * _NS
_SPW = _BLOCK // _NW


def _emb_body(x_hbm, tok_hbm, pos_hbm, out_hbm, idx_v, rows_v, pos_v,
              s0, s1, s2, s3, sp, so):
    gsems = [s0, s1, s2, s3]
    wid = lax.axis_index("s") * _NC + lax.axis_index("c")
    base = wid * _SPW
    pcp = pltpu.async_copy(pos_hbm.at[pl.ds(base, _SPW)], pos_v, sp)
    icps = [
        pltpu.async_copy(x_hbm.at[b, pl.ds(base, _SPW)], idx_v.at[b],
                         gsems[b])
        for b in range(_BATCH)
    ]
    gcps = []
    for b in range(_BATCH):
        icps[b].wait()
        gcps.append(
            pltpu.async_copy(tok_hbm.at[idx_v.at[b]], rows_v.at[b], gsems[b])
        )
    pcp.wait()
    ocps = []
    for b in range(_BATCH):
        gcps[b].wait()

        @plsc.parallel_loop(0, _SPW, unroll=2)
        def add_row(i, b=b):
            for c in range(_EMBED // _L):
                s = pl.ds(c * _L, _L)
                plsc.addupdate(rows_v.at[b, i, s], pos_v[i, s])
        ocps.append(
            pltpu.async_copy(rows_v.at[b], out_hbm.at[b, pl.ds(base, _SPW)],
                             so)
        )
    for cp in ocps:
        cp.wait()


@jax.jit
def _emb(x, tok_table, pos_table):
    mesh = plsc.VectorSubcoreMesh(core_axis_name="c", subcore_axis_name="s")
    k = functools.partial(
        pl.kernel,
        mesh=mesh,
        out_type=jax.ShapeDtypeStruct((_BATCH, _BLOCK, _EMBED), jnp.float32),
        scratch_types=[
            pltpu.VMEM((_BATCH, _SPW), jnp.int32),
            pltpu.VMEM((_BATCH, _SPW, _EMBED), jnp.float32),
            pltpu.VMEM((_SPW, _EMBED), jnp.float32),
        ] + [pltpu.SemaphoreType.DMA] * 6,
    )(_emb_body)
    return k(x, tok_table, pos_table)


def kernel(x, tok_table, pos_table):
    return _emb(x.astype(jnp.int32), tok_table, pos_table)


# speedup vs baseline: 1.0140x; 1.0140x over previous
import functools

import jax
import jax.numpy as jnp
from jax import lax
from jax.experimental import pallas as pl
from jax.experimental.pallas import tpu as pltpu
from jax.experimental.pallas import tpu_sc as plsc

_EMBED = 128
_BLOCK = 2048
_BATCH = 4

_info = plsc.get_sparse_core_info()
_NC, _NS, _L = _info.num_cores, _info.num_subcores, _info.num_lanes
_NW = _NC * _NS
_SPW = _BLOCK // _NW


_HC = 2
_CR = _SPW // _HC


def _emb_body(x_hbm, tok_hbm, pos_hbm, out_hbm, idx_v, rows_v, pos_v,
              s0, s1, s2, s3, s4, s5, s6, s7, sp, so):
    gsems = [s0, s1, s2, s3, s4, s5, s6, s7]
    wid = lax.axis_index("s") * _NC + lax.axis_index("c")
    base = wid * _SPW
    pcp = pltpu.async_copy(pos_hbm.at[pl.ds(base, _SPW)], pos_v, sp)
    icps = [
        pltpu.async_copy(x_hbm.at[b, pl.ds(base, _SPW)], idx_v.at[b],
                         gsems[2 * b])
        for b in range(_BATCH)
    ]
    gcps = []
    for b in range(_BATCH):
        icps[b].wait()
        for h in range(_HC):
            k = b * _HC + h
            gcps.append(
                pltpu.async_copy(
                    tok_hbm.at[idx_v.at[b, pl.ds(h * _CR, _CR)]],
                    rows_v.at[b, pl.ds(h * _CR, _CR)],
                    gsems[k],
                )
            )
    pcp.wait()
    ocps = []
    for b in range(_BATCH):
        for h in range(_HC):
            k = b * _HC + h
            gcps[k].wait()

            def add_row(i, carry, b=b, h=h):
                for c in range(_EMBED // _L):
                    s = pl.ds(c * _L, _L)
                    plsc.addupdate(rows_v.at[b, h * _CR + i, s],
                                   pos_v[h * _CR + i, s])
                return carry

            lax.fori_loop(0, _CR, add_row, 0)
            ocps.append(
                pltpu.async_copy(
                    rows_v.at[b, pl.ds(h * _CR, _CR)],
                    out_hbm.at[b, pl.ds(base + h * _CR, _CR)],
                    so,
                )
            )
    for cp in ocps:
        cp.wait()


@jax.jit
def _emb(x, tok_table, pos_table):
    mesh = plsc.VectorSubcoreMesh(core_axis_name="c", subcore_axis_name="s")
    k = functools.partial(
        pl.kernel,
        mesh=mesh,
        out_type=jax.ShapeDtypeStruct((_BATCH, _BLOCK, _EMBED), jnp.float32),
        scratch_types=[
            pltpu.VMEM((_BATCH, _SPW), jnp.int32),
            pltpu.VMEM((_BATCH, _SPW, _EMBED), jnp.float32),
            pltpu.VMEM((_SPW, _EMBED), jnp.float32),
        ] + [pltpu.SemaphoreType.DMA] * 10,
    )(_emb_body)
    return k(x, tok_table, pos_table)


def kernel(x, tok_table, pos_table):
    return _emb(x.astype(jnp.int32), tok_table, pos_table)


# speedup vs baseline: 1.0154x; 1.0013x over previous
import functools

import jax
import jax.numpy as jnp
from jax import lax
from jax.experimental import pallas as pl
from jax.experimental.pallas import tpu as pltpu
from jax.experimental.pallas import tpu_sc as plsc

_EMBED = 128
_BLOCK = 2048
_BATCH = 4

_info = plsc.get_sparse_core_info()
_NC, _NS, _L = _info.num_cores, _info.num_subcores, _info.num_lanes
_NW = _NC * _NS
_SPW = _BLOCK // _NW
_HC = 2
_CR = _SPW // _HC


def _emb_body(x_hbm, tok_hbm, pos_hbm, out_hbm, idx_v, buf_v, gsem, sp, so):
    rows_v = buf_v
    wid = lax.axis_index("s") * _NC + lax.axis_index("c")
    base = wid * _SPW
    pcp = pltpu.async_copy(pos_hbm.at[pl.ds(base, _SPW)],
                           buf_v.at[_BATCH], sp)
    icps = [
        pltpu.async_copy(x_hbm.at[b, pl.ds(base, _SPW)], idx_v.at[b],
                         gsem.at[2 * b])
        for b in range(_BATCH)
    ]
    gcps = []
    for b in range(_BATCH):
        icps[b].wait()
        for h in range(_HC):
            k = b * _HC + h
            gcps.append(
                pltpu.async_copy(
                    tok_hbm.at[idx_v.at[b, pl.ds(h * _CR, _CR)]],
                    rows_v.at[b, pl.ds(h * _CR, _CR)],
                    gsem.at[k],
                )
            )
    pcp.wait()
    ocps = []
    for b in range(_BATCH):
        for h in range(_HC):
            k = b * _HC + h
            gcps[k].wait()

            def add_row(i, carry, b=b, h=h):
                for c in range(_EMBED // _L):
                    s = pl.ds(c * _L, _L)
                    plsc.addupdate(rows_v.at[b, h * _CR + i, s],
                                   buf_v[_BATCH, h * _CR + i, s])
                return carry

            lax.fori_loop(0, _CR, add_row, 0)
            ocps.append(
                pltpu.async_copy(
                    rows_v.at[b, pl.ds(h * _CR, _CR)],
                    out_hbm.at[b, pl.ds(base + h * _CR, _CR)],
                    so,
                )
            )
    for cp in ocps:
        cp.wait()


@jax.jit
def _emb(x, tok_table, pos_table):
    mesh = plsc.VectorSubcoreMesh(core_axis_name="c", subcore_axis_name="s")
    k = functools.partial(
        pl.kernel,
        mesh=mesh,
        out_type=jax.ShapeDtypeStruct((_BATCH, _BLOCK, _EMBED), jnp.float32),
        scratch_types=[
            pltpu.VMEM((_BATCH, _SPW), jnp.int32),
            pltpu.VMEM((_BATCH + 1, _SPW, _EMBED), jnp.float32),
            pltpu.SemaphoreType.DMA((_BATCH * _HC,)),
            pltpu.SemaphoreType.DMA,
            pltpu.SemaphoreType.DMA,
        ],
    )(_emb_body)
    return k(x, tok_table, pos_table)


def kernel(x, tok_table, pos_table):
    return _emb(x.astype(jnp.int32), tok_table, pos_table)
